# baseline (speedup 1.0000x reference)
import jax
import jax.numpy as jnp
from jax.experimental import pallas as pl
from jax.experimental.pallas import tpu as pltpu

_B = 256
_WIN = 128
_FEAT = 128
_EMB = 128
_HEADS = 4
_EPG = 2048
_NODE = 128
_G = 8
_F32 = jnp.float32
_BF16 = jnp.bfloat16


def _dot(a, b, dims, out=_F32):
    return jax.lax.dot_general(a, b, (dims, ((), ())),
                               preferred_element_type=out)


def _body(x_ref, src_ref, dst_ref,
          wih_ref, whh_ref, bias_ref,
          wl_ref, bl_ref, wr_ref, br_ref, att_ref, attbd_ref, gb_ref,
          wq_ref, bq_ref, wk_ref, bk_ref, wv_ref, bv_ref,
          wo_ref, bo_ref, w1_ref, b1_ref, w2_ref, b2_ref,
          recon_ref, fc_ref, alpha_ref,
          ht_ref, sd16_ref, s16_ref, d16_ref, s32_ref, d32_ref):
    @pl.when(pl.program_id(0) == 0)
    def _():
        ht_ref[...] = jnp.zeros_like(ht_ref)
        lane = jax.lax.broadcasted_iota(jnp.int32, (_EPG, _NODE), 1)
        s = (src_ref[...] == lane)
        d = (dst_ref[...] == lane)
        s16_ref[...] = s.astype(_BF16)
        d16_ref[...] = d.astype(_BF16)
        sd16_ref[...] = jnp.concatenate(
            [s.astype(_BF16), d.astype(_BF16)], axis=1)
        s32_ref[...] = s.astype(_F32)
        d32_ref[...] = d.astype(_F32)

    xcat = jnp.concatenate([x_ref[s] for s in range(_G)], axis=1)
    c = _dot(wih_ref[...], xcat, ((1,), (0,))) + bias_ref[...]
    h = ht_ref[...]
    ys = []
    for s in range(_G):
        pre = _dot(whh_ref[...], h.astype(_BF16), ((1,), (0,)))
        h = jnp.tanh(c[:, s * _FEAT:(s + 1) * _FEAT] + pre)
        ys.append(h)
    ht_ref[...] = h
    yTw = jnp.concatenate(ys, axis=1)

    xlTw = _dot(wl_ref[...], yTw, ((1,), (0,))) + bl_ref[...]
    xrTw = _dot(wr_ref[...], yTw, ((1,), (0,))) + br_ref[...]
    alm = _dot(att_ref[...], xlTw, ((1,), (0,))).reshape(_G, _NODE)
    arm = _dot(att_ref[...], xrTw, ((1,), (0,))).reshape(_G, _NODE)
    xlTm = jnp.concatenate(
        [xlTw[:, g * _NODE:(g + 1) * _NODE] for g in range(_G)],
        axis=0).astype(_BF16)
    xrTm = jnp.concatenate(
        [xrTw[:, g * _NODE:(g + 1) * _NODE] for g in range(_G)],
        axis=0).astype(_BF16)

    S16 = s16_ref[...]
    D16 = d16_ref[...]
    S32 = s32_ref[...]
    D32 = d32_ref[...]

    XLR = jnp.concatenate([xlTm, xrTm], axis=1)
    T = _dot(sd16_ref[...], XLR, ((1,), (1,)))
    Aabs = jnp.abs(T).astype(_BF16)

    lin = (_dot(S32, alm, ((1,), (1,))) +
           _dot(D32, arm, ((1,), (1,))))
    e = 0.6 * lin + 0.4 * _dot(Aabs, attbd_ref[...], ((1,), (0,)))

    gmax = jnp.max(e, axis=0, keepdims=True)
    ex = jnp.exp(e - gmax)
    denom = _dot(D32, ex, ((0,), (0,)))
    rec = 1.0 / jnp.maximum(denom, 1e-16)
    alpha = ex * _dot(D32, rec, ((1,), (0,)))
    alpha_ref[...] = alpha.reshape(1, _EPG, _G)

    ab = alpha.astype(_BF16)
    zs = []
    for g in range(_G):
        dsc = D16 * ab[:, g:g + 1]
        wg = _dot(dsc, S16, ((0,), (0,))).astype(_BF16)
        z = _dot(wg, xlTm[g * _EMB:(g + 1) * _EMB, :], ((1,), (1,)))
        zs.append(z + gb_ref[...])
    zm = jnp.concatenate(zs, axis=0)
    zb = zm.astype(_BF16)
    zwb = jnp.concatenate(zs, axis=1).astype(_BF16)

    q = (_dot(zb, wq_ref[...], ((1,), (1,))) + bq_ref[...]).astype(_BF16)
    k = (_dot(zb, wk_ref[...], ((1,), (1,))) + bk_ref[...]).astype(_BF16)
    v = (_dot(zb, wv_ref[...], ((1,), (1,))) + bv_ref[...]).astype(_BF16)
    dh = _FEAT // _HEADS
    scale = 1.0 / (dh ** 0.5)
    om = []
    for g in range(_G):
        outs = []
        for h in range(_HEADS):
            qh = q[g * _NODE:(g + 1) * _NODE, h * dh:(h + 1) * dh]
            kh = k[g * _NODE:(g + 1) * _NODE, h * dh:(h + 1) * dh]
            vh = v[g * _NODE:(g + 1) * _NODE, h * dh:(h + 1) * dh]
            p = jnp.exp(_dot(qh, kh, ((1,), (1,))) * scale)
            sm = jnp.sum(p, axis=1, keepdims=True)
            oh = _dot(p.astype(_BF16), vh, ((1,), (0,))) / sm
            outs.append(oh.astype(_BF16))
        om.append(jnp.concatenate(outs, axis=1))
    omc = jnp.concatenate(om, axis=0)
    recon_ref[...] = (_dot(omc, wo_ref[...], ((1,), (1,)))
                      + bo_ref[...]).reshape(1, _G * _NODE, _FEAT)

    hm = jnp.maximum(_dot(w1_ref[...], zwb, ((1,), (0,))) + b1_ref[...],
                     0.0)
    fc_ref[...] = (_dot(w2_ref[...], hm.astype(_BF16), ((1,), (0,)))
                   + b2_ref[...]).reshape(1, 1, _G * _FEAT)


def _full(shape):
    return pl.BlockSpec(shape, lambda b: (0,) * len(shape))


def kernel(x, edge_idx, params):
    p = params
    src_col = edge_idx[0, 0].reshape(_EPG, 1).astype(jnp.int32)
    dst_col = edge_idx[0, 1].reshape(_EPG, 1).astype(jnp.int32)
    bias_col = (p['b_ih'] + p['b_hh']).reshape(_EMB, 1)
    att_bd = jnp.kron(jnp.eye(_G, dtype=_F32),
                      p['att'].reshape(_EMB, 1)).astype(_BF16)

    wb = lambda w: w.astype(_BF16)
    recon, fc, alpha = pl.pallas_call(
        _body,
        grid=(_B // _G,),
        in_specs=[
            pl.BlockSpec((_G, _WIN, _FEAT), lambda b: (b, 0, 0)),
            _full((_EPG, 1)),
            _full((_EPG, 1)),
            _full((_EMB, _WIN)),
            _full((_EMB, _EMB)),
            _full((_EMB, 1)),
            _full((_EMB, _EMB)),
            _full((_EMB, 1)),
            _full((_EMB, _EMB)),
            _full((_EMB, 1)),
            _full((1, _EMB)),
            _full((_G * _EMB, _G)),
            _full((1, _EMB)),
            _full((_FEAT, _FEAT)),
            _full((1, _FEAT)),
            _full((_FEAT, _FEAT)),
            _full((1, _FEAT)),
            _full((_FEAT, _FEAT)),
            _full((1, _FEAT)),
            _full((_FEAT, _FEAT)),
            _full((1, _FEAT)),
            _full((_EMB, _EMB)),
            _full((_EMB, 1)),
            _full((1, _EMB)),
            _full((1, 1)),
        ],
        out_specs=[
            pl.BlockSpec((1, _G * _NODE, _FEAT), lambda b: (b, 0, 0)),
            pl.BlockSpec((1, 1, _G * _FEAT), lambda b: (b, 0, 0)),
            pl.BlockSpec((1, _EPG, _G), lambda b: (b, 0, 0)),
        ],
        out_shape=[
            jax.ShapeDtypeStruct((_B // _G, _G * _NODE, _FEAT), _F32),
            jax.ShapeDtypeStruct((_B // _G, 1, _G * _FEAT), _F32),
            jax.ShapeDtypeStruct((_B // _G, _EPG, _G), _F32),
        ],
        scratch_shapes=[
            pltpu.VMEM((_EMB, _FEAT), _F32),
            pltpu.VMEM((_EPG, 2 * _NODE), _BF16),
            pltpu.VMEM((_EPG, _NODE), _BF16),
            pltpu.VMEM((_EPG, _NODE), _BF16),
            pltpu.VMEM((_EPG, _NODE), _F32),
            pltpu.VMEM((_EPG, _NODE), _F32),
        ],
        compiler_params=pltpu.CompilerParams(
            dimension_semantics=("arbitrary",)),
    )(x, src_col, dst_col,
      p['W_ih'], wb(p['W_hh']), bias_col,
      p['Wl'], p['bl'].reshape(_EMB, 1),
      p['Wr'], p['br'].reshape(_EMB, 1),
      p['att'].reshape(1, _EMB), att_bd, p['gat_bias'].reshape(1, _EMB),
      wb(p['Wq']), p['bq'].reshape(1, _FEAT),
      wb(p['Wk']), p['bk'].reshape(1, _FEAT),
      wb(p['Wv']), p['bv'].reshape(1, _FEAT),
      wb(p['Wo']), p['bo'].reshape(1, _FEAT),
      wb(p['W1']), p['b1'].reshape(_EMB, 1),
      wb(p['W2']), p['b2'].reshape(1, 1))

    return (recon.reshape(_B, _EMB, _FEAT), fc.reshape(_B, _FEAT),
            alpha.transpose(0, 2, 1).reshape(_B * _EPG))


# speedup vs baseline: 1.0603x; 1.0603x over previous
import jax
import jax.numpy as jnp
from jax.experimental import pallas as pl
from jax.experimental.pallas import tpu as pltpu

_B = 256
_WIN = 128
_FEAT = 128
_EMB = 128
_HEADS = 4
_EPG = 2048
_NODE = 128
_G = 16
_F32 = jnp.float32
_BF16 = jnp.bfloat16


def _dot(a, b, dims, out=_F32):
    return jax.lax.dot_general(a, b, (dims, ((), ())),
                               preferred_element_type=out)


def _body(x_ref, src_ref, dst_ref,
          wih_ref, whh_ref, bias_ref,
          wl_ref, bl_ref, wr_ref, br_ref, att_ref, attbd_ref, gb_ref,
          wq_ref, bq_ref, wk_ref, bk_ref, wv_ref, bv_ref,
          wo_ref, bo_ref, w1_ref, b1_ref, w2_ref, b2_ref,
          recon_ref, fc_ref, alpha_ref,
          ht_ref, sd16_ref, s16_ref, d16_ref, s32_ref, d32_ref):
    @pl.when(pl.program_id(0) == 0)
    def _():
        ht_ref[...] = jnp.zeros_like(ht_ref)
        lane = jax.lax.broadcasted_iota(jnp.int32, (_EPG, _NODE), 1)
        s = (src_ref[...] == lane)
        d = (dst_ref[...] == lane)
        s16_ref[...] = s.astype(_BF16)
        d16_ref[...] = d.astype(_BF16)
        sd16_ref[...] = jnp.concatenate(
            [s.astype(_BF16), d.astype(_BF16)], axis=1)
        s32_ref[...] = s.astype(_F32)
        d32_ref[...] = d.astype(_F32)

    xcat = jnp.concatenate([x_ref[s] for s in range(_G)], axis=1)
    c = _dot(wih_ref[...], xcat, ((1,), (0,))) + bias_ref[...]
    h = ht_ref[...]
    ys = []
    for s in range(_G):
        pre = _dot(whh_ref[...], h.astype(_BF16), ((1,), (0,)))
        h = jnp.tanh(c[:, s * _FEAT:(s + 1) * _FEAT] + pre)
        ys.append(h)
    ht_ref[...] = h
    yTw = jnp.concatenate(ys, axis=1)

    xlTw = _dot(wl_ref[...], yTw, ((1,), (0,))) + bl_ref[...]
    xrTw = _dot(wr_ref[...], yTw, ((1,), (0,))) + br_ref[...]
    alm = _dot(att_ref[...], xlTw, ((1,), (0,))).reshape(_G, _NODE)
    arm = _dot(att_ref[...], xrTw, ((1,), (0,))).reshape(_G, _NODE)
    xlTm = jnp.concatenate(
        [xlTw[:, g * _NODE:(g + 1) * _NODE] for g in range(_G)],
        axis=0).astype(_BF16)
    xrTm = jnp.concatenate(
        [xrTw[:, g * _NODE:(g + 1) * _NODE] for g in range(_G)],
        axis=0).astype(_BF16)

    S16 = s16_ref[...]
    D16 = d16_ref[...]
    S32 = s32_ref[...]
    D32 = d32_ref[...]

    XLR = jnp.concatenate([xlTm, xrTm], axis=1)
    T = _dot(sd16_ref[...], XLR, ((1,), (1,)))
    Aabs = jnp.abs(T).astype(_BF16)

    lin = (_dot(S32, alm, ((1,), (1,))) +
           _dot(D32, arm, ((1,), (1,))))
    e = 0.6 * lin + 0.4 * _dot(Aabs, attbd_ref[...], ((1,), (0,)))

    gmax = jnp.max(e, axis=0, keepdims=True)
    ex = jnp.exp(e - gmax)
    denom = _dot(D32, ex, ((0,), (0,)))
    rec = 1.0 / jnp.maximum(denom, 1e-16)
    alpha = ex * _dot(D32, rec, ((1,), (0,)))
    alpha_ref[...] = alpha.reshape(1, _EPG, _G)

    ab = alpha.astype(_BF16)
    zs = []
    for g in range(_G):
        dsc = D16 * ab[:, g:g + 1]
        wg = _dot(dsc, S16, ((0,), (0,))).astype(_BF16)
        z = _dot(wg, xlTm[g * _EMB:(g + 1) * _EMB, :], ((1,), (1,)))
        zs.append(z + gb_ref[...])
    zm = jnp.concatenate(zs, axis=0)
    zb = zm.astype(_BF16)
    zwb = jnp.concatenate(zs, axis=1).astype(_BF16)

    q = (_dot(zb, wq_ref[...], ((1,), (1,))) + bq_ref[...]).astype(_BF16)
    k = (_dot(zb, wk_ref[...], ((1,), (1,))) + bk_ref[...]).astype(_BF16)
    v = (_dot(zb, wv_ref[...], ((1,), (1,))) + bv_ref[...]).astype(_BF16)
    dh = _FEAT // _HEADS
    scale = 1.0 / (dh ** 0.5)
    om = []
    for g in range(_G):
        outs = []
        for h in range(_HEADS):
            qh = q[g * _NODE:(g + 1) * _NODE, h * dh:(h + 1) * dh]
            kh = k[g * _NODE:(g + 1) * _NODE, h * dh:(h + 1) * dh]
            vh = v[g * _NODE:(g + 1) * _NODE, h * dh:(h + 1) * dh]
            p = jnp.exp(_dot(qh, kh, ((1,), (1,))) * scale)
            sm = jnp.sum(p, axis=1, keepdims=True)
            oh = _dot(p.astype(_BF16), vh, ((1,), (0,))) / sm
            outs.append(oh.astype(_BF16))
        om.append(jnp.concatenate(outs, axis=1))
    omc = jnp.concatenate(om, axis=0)
    recon_ref[...] = (_dot(omc, wo_ref[...], ((1,), (1,)))
                      + bo_ref[...]).reshape(1, _G * _NODE, _FEAT)

    hm = jnp.maximum(_dot(w1_ref[...], zwb, ((1,), (0,))) + b1_ref[...],
                     0.0)
    fc_ref[...] = (_dot(w2_ref[...], hm.astype(_BF16), ((1,), (0,)))
                   + b2_ref[...]).reshape(1, 1, _G * _FEAT)


def _full(shape):
    return pl.BlockSpec(shape, lambda b: (0,) * len(shape))


def kernel(x, edge_idx, params):
    p = params
    src_col = edge_idx[0, 0].reshape(_EPG, 1).astype(jnp.int32)
    dst_col = edge_idx[0, 1].reshape(_EPG, 1).astype(jnp.int32)
    bias_col = (p['b_ih'] + p['b_hh']).reshape(_EMB, 1)
    att_bd = jnp.kron(jnp.eye(_G, dtype=_F32),
                      p['att'].reshape(_EMB, 1)).astype(_BF16)

    wb = lambda w: w.astype(_BF16)
    recon, fc, alpha = pl.pallas_call(
        _body,
        grid=(_B // _G,),
        in_specs=[
            pl.BlockSpec((_G, _WIN, _FEAT), lambda b: (b, 0, 0)),
            _full((_EPG, 1)),
            _full((_EPG, 1)),
            _full((_EMB, _WIN)),
            _full((_EMB, _EMB)),
            _full((_EMB, 1)),
            _full((_EMB, _EMB)),
            _full((_EMB, 1)),
            _full((_EMB, _EMB)),
            _full((_EMB, 1)),
            _full((1, _EMB)),
            _full((_G * _EMB, _G)),
            _full((1, _EMB)),
            _full((_FEAT, _FEAT)),
            _full((1, _FEAT)),
            _full((_FEAT, _FEAT)),
            _full((1, _FEAT)),
            _full((_FEAT, _FEAT)),
            _full((1, _FEAT)),
            _full((_FEAT, _FEAT)),
            _full((1, _FEAT)),
            _full((_EMB, _EMB)),
            _full((_EMB, 1)),
            _full((1, _EMB)),
            _full((1, 1)),
        ],
        out_specs=[
            pl.BlockSpec((1, _G * _NODE, _FEAT), lambda b: (b, 0, 0)),
            pl.BlockSpec((1, 1, _G * _FEAT), lambda b: (b, 0, 0)),
            pl.BlockSpec((1, _EPG, _G), lambda b: (b, 0, 0)),
        ],
        out_shape=[
            jax.ShapeDtypeStruct((_B // _G, _G * _NODE, _FEAT), _F32),
            jax.ShapeDtypeStruct((_B // _G, 1, _G * _FEAT), _F32),
            jax.ShapeDtypeStruct((_B // _G, _EPG, _G), _F32),
        ],
        scratch_shapes=[
            pltpu.VMEM((_EMB, _FEAT), _F32),
            pltpu.VMEM((_EPG, 2 * _NODE), _BF16),
            pltpu.VMEM((_EPG, _NODE), _BF16),
            pltpu.VMEM((_EPG, _NODE), _BF16),
            pltpu.VMEM((_EPG, _NODE), _F32),
            pltpu.VMEM((_EPG, _NODE), _F32),
        ],
        compiler_params=pltpu.CompilerParams(
            dimension_semantics=("arbitrary",)),
    )(x, src_col, dst_col,
      p['W_ih'], wb(p['W_hh']), bias_col,
      p['Wl'], p['bl'].reshape(_EMB, 1),
      p['Wr'], p['br'].reshape(_EMB, 1),
      p['att'].reshape(1, _EMB), att_bd, p['gat_bias'].reshape(1, _EMB),
      wb(p['Wq']), p['bq'].reshape(1, _FEAT),
      wb(p['Wk']), p['bk'].reshape(1, _FEAT),
      wb(p['Wv']), p['bv'].reshape(1, _FEAT),
      wb(p['Wo']), p['bo'].reshape(1, _FEAT),
      wb(p['W1']), p['b1'].reshape(_EMB, 1),
      wb(p['W2']), p['b2'].reshape(1, 1))

    return (recon.reshape(_B, _EMB, _FEAT), fc.reshape(_B, _FEAT),
            alpha.transpose(0, 2, 1).reshape(_B * _EPG))
